# Initial kernel scaffold; baseline (speedup 1.0000x reference)
import jax
import jax.numpy as jnp
from jax.experimental import pallas as pl


def kernel(hidden_states, W_router, W_gate, W_up, W_down):
    raise NotImplementedError("write your pallas kernel here")


# speedup vs baseline: 1.3814x; 1.3814x over previous
import functools

import jax
import jax.numpy as jnp
from jax import lax
from jax.experimental import pallas as pl
from jax.experimental.pallas import tpu as pltpu
from jax.experimental.pallas import tpu_sc as plsc

_K = 2
_BLK = 256
_IB = 1024
_TBR = 512
_TBS = 512
_NW = 32
_CH = 64


def _router(x, W_router):
    T, H = x.shape
    E = W_router.shape[1]
    nb = T // _TBR

    def body(x_ref, wr_ref, id0_ref, id1_ref, w0_ref, w1_ref, loss_ref, acc_ref):
        i = pl.program_id(0)
        logits = jnp.dot(x_ref[...], wr_ref[...], preferred_element_type=jnp.float32)
        m = jnp.max(logits, axis=-1, keepdims=True)
        ex = jnp.exp(logits - m)
        p = ex / jnp.sum(ex, axis=-1, keepdims=True)
        iota = lax.broadcasted_iota(jnp.int32, p.shape, 1)
        m1 = jnp.max(p, axis=-1, keepdims=True)
        id0 = jnp.min(jnp.where(p == m1, iota, E), axis=-1, keepdims=True)
        p2 = jnp.where(iota == id0, -1.0, p)
        m2 = jnp.max(p2, axis=-1, keepdims=True)
        id1 = jnp.min(jnp.where(p2 == m2, iota, E), axis=-1, keepdims=True)
        s = m1 + m2
        id0_ref[...] = id0
        id1_ref[...] = id1
        w0_ref[...] = m1 / s
        w1_ref[...] = m2 / s
        pad = jnp.zeros((1, 128 - E), jnp.float32)
        psum = jnp.concatenate([jnp.sum(p, axis=0, keepdims=True), pad], axis=1)
        hit = (iota == id0).astype(jnp.float32) + (iota == id1).astype(jnp.float32)
        cnt = jnp.concatenate([jnp.sum(hit, axis=0, keepdims=True), pad], axis=1)

        @pl.when(i == 0)
        def _():
            acc_ref[...] = jnp.zeros_like(acc_ref)

        acc_ref[0:1, :] += psum
        acc_ref[1:2, :] += cnt

        @pl.when(i == nb - 1)
        def _():
            loss_ref[0, 0] = (jnp.sum(acc_ref[0:1, :] * acc_ref[1:2, :])
                              * E / (T * T))

    return pl.pallas_call(
        body,
        grid=(nb,),
        in_specs=[
            pl.BlockSpec((_TBR, H), lambda i: (i, 0)),
            pl.BlockSpec((H, E), lambda i: (0, 0)),
        ],
        out_specs=[
            pl.BlockSpec((_TBR, 1), lambda i: (i, 0)),
            pl.BlockSpec((_TBR, 1), lambda i: (i, 0)),
            pl.BlockSpec((_TBR, 1), lambda i: (i, 0)),
            pl.BlockSpec((_TBR, 1), lambda i: (i, 0)),
            pl.BlockSpec((1, 1), lambda i: (0, 0), memory_space=pltpu.SMEM),
        ],
        out_shape=[
            jax.ShapeDtypeStruct((T, 1), jnp.int32),
            jax.ShapeDtypeStruct((T, 1), jnp.int32),
            jax.ShapeDtypeStruct((T, 1), jnp.float32),
            jax.ShapeDtypeStruct((T, 1), jnp.float32),
            jax.ShapeDtypeStruct((1, 1), jnp.float32),
        ],
        scratch_shapes=[pltpu.VMEM((8, 128), jnp.float32)],
    )(x, W_router)


def _sc_gather_rows(table, idx):
    R = idx.shape[0]
    H = table.shape[1]
    per = R // _NW
    nch = per // _CH
    mesh = plsc.VectorSubcoreMesh(core_axis_name="c", subcore_axis_name="s")

    @functools.partial(
        pl.kernel,
        out_type=jax.ShapeDtypeStruct((R, H), jnp.float32),
        mesh=mesh,
        scratch_types=[
            pltpu.VMEM((_CH,), jnp.int32),
            pltpu.VMEM((_CH, H), jnp.float32),
            pltpu.SemaphoreType.DMA,
        ],
    )
    def k(idx_hbm, tab_hbm, out_hbm, idx_v, rows_v, sem):
        wid = lax.axis_index("s") * 2 + lax.axis_index("c")
        for c in range(nch):
            base = wid * per + c * _CH
            pltpu.sync_copy(idx_hbm.at[pl.ds(base, _CH)], idx_v)
            pltpu.async_copy(tab_hbm.at[idx_v], rows_v, sem).wait()
            pltpu.sync_copy(rows_v, out_hbm.at[pl.ds(base, _CH)])

    return k(idx, table)


def _grouped_ffn(x_sorted, gate_pad, block_expert, W_gate, W_up, W_down):
    R, H = x_sorted.shape
    E, _, I = W_gate.shape
    G = R // _BLK
    KC = I // _IB

    def body(ids_ref, x_ref, gate_ref, wg_ref, wu_ref, wd_ref, y_ref):
        kc = pl.program_id(1)
        x = x_ref[...]
        g = jnp.dot(x, wg_ref[0], preferred_element_type=jnp.float32)
        u = jnp.dot(x, wu_ref[0], preferred_element_type=jnp.float32)
        a = g * jax.nn.sigmoid(g) * u
        part = jnp.dot(a, wd_ref[0], preferred_element_type=jnp.float32)
        part = part * gate_ref[...]

        @pl.when(kc == 0)
        def _():
            y_ref[...] = part

        @pl.when(kc > 0)
        def _():
            y_ref[...] += part

    grid_spec = pltpu.PrefetchScalarGridSpec(
        num_scalar_prefetch=1,
        grid=(G, KC),
        in_specs=[
            pl.BlockSpec((_BLK, H), lambda g, kc, ids: (g, 0)),
            pl.BlockSpec((_BLK, 1), lambda g, kc, ids: (g, 0)),
            pl.BlockSpec((1, H, _IB), lambda g, kc, ids: (ids[g], 0, kc)),
            pl.BlockSpec((1, H, _IB), lambda g, kc, ids: (ids[g], 0, kc)),
            pl.BlockSpec((1, _IB, H), lambda g, kc, ids: (ids[g], kc, 0)),
        ],
        out_specs=pl.BlockSpec((_BLK, H), lambda g, kc, ids: (g, 0)),
    )
    return pl.pallas_call(
        body,
        grid_spec=grid_spec,
        out_shape=jax.ShapeDtypeStruct((R, H), jnp.float32),
    )(block_expert, x_sorted, gate_pad, W_gate, W_up, W_down)


def _pair_sum(combined, T):
    H = combined.shape[1]
    nb = T // _TBS

    def body(a_ref, b_ref, o_ref):
        o_ref[...] = a_ref[...] + b_ref[...]

    return pl.pallas_call(
        body,
        grid=(nb,),
        in_specs=[
            pl.BlockSpec((_TBS, H), lambda i: (i, 0)),
            pl.BlockSpec((_TBS, H), lambda i: (i + nb, 0)),
        ],
        out_specs=pl.BlockSpec((_TBS, H), lambda i: (i, 0)),
        out_shape=jax.ShapeDtypeStruct((T, H), jnp.float32),
    )(combined, combined)


def kernel(hidden_states, W_router, W_gate, W_up, W_down):
    B, S, H = hidden_states.shape
    E = W_router.shape[1]
    T = B * S
    A = _K * T
    G = A // _BLK + E
    R = G * _BLK

    x = hidden_states.reshape(T, H)
    id0, id1, w0, w1, loss = _router(x, W_router)

    e_flat = jnp.concatenate([id0[:, 0], id1[:, 0]])
    gate_flat = jnp.concatenate([w0[:, 0], w1[:, 0]])
    order = jnp.argsort(e_flat)
    e_sorted = e_flat[order]
    counts = jnp.bincount(e_flat, length=E)
    nrows_pad = ((counts + _BLK - 1) // _BLK) * _BLK
    zero = jnp.zeros((1,), counts.dtype)
    pstart = jnp.concatenate([zero, jnp.cumsum(nrows_pad)])[:E]
    start = jnp.concatenate([zero, jnp.cumsum(counts)])[:E]
    pp = (pstart[e_sorted] + jnp.arange(A) - start[e_sorted]).astype(jnp.int32)
    tok_pad = jnp.zeros((R,), jnp.int32).at[pp].set((order % T).astype(jnp.int32))
    gate_pad = jnp.zeros((R, 1), jnp.float32).at[pp, 0].set(gate_flat[order])
    src = jnp.zeros((A,), jnp.int32).at[order].set(pp)
    bstart = pstart // _BLK
    block_expert = (jnp.sum(jnp.arange(G)[:, None] >= bstart[None, :], axis=1)
                    .astype(jnp.int32) - 1)

    x_sorted = _sc_gather_rows(x, tok_pad)
    y_pad = _grouped_ffn(x_sorted, gate_pad, block_expert, W_gate, W_up, W_down)
    combined = _sc_gather_rows(y_pad, src)
    out = _pair_sum(combined, T).reshape(B, S, H)
    return out, loss[0, 0]


# speedup vs baseline: 1.3869x; 1.0040x over previous
import functools

import jax
import jax.numpy as jnp
from jax import lax
from jax.experimental import pallas as pl
from jax.experimental.pallas import tpu as pltpu
from jax.experimental.pallas import tpu_sc as plsc

_K = 2
_BLK = 256
_IB = 1024
_TBR = 512
_TBS = 512
_NW = 32
_CH = 32


def _router(x, W_router):
    T, H = x.shape
    E = W_router.shape[1]
    nb = T // _TBR

    def body(x_ref, wr_ref, id0_ref, id1_ref, w0_ref, w1_ref, loss_ref, acc_ref):
        i = pl.program_id(0)
        logits = jnp.dot(x_ref[...], wr_ref[...], preferred_element_type=jnp.float32)
        m = jnp.max(logits, axis=-1, keepdims=True)
        ex = jnp.exp(logits - m)
        p = ex / jnp.sum(ex, axis=-1, keepdims=True)
        iota = lax.broadcasted_iota(jnp.int32, p.shape, 1)
        m1 = jnp.max(p, axis=-1, keepdims=True)
        id0 = jnp.min(jnp.where(p == m1, iota, E), axis=-1, keepdims=True)
        p2 = jnp.where(iota == id0, -1.0, p)
        m2 = jnp.max(p2, axis=-1, keepdims=True)
        id1 = jnp.min(jnp.where(p2 == m2, iota, E), axis=-1, keepdims=True)
        s = m1 + m2
        id0_ref[...] = id0
        id1_ref[...] = id1
        w0_ref[...] = m1 / s
        w1_ref[...] = m2 / s
        pad = jnp.zeros((1, 128 - E), jnp.float32)
        psum = jnp.concatenate([jnp.sum(p, axis=0, keepdims=True), pad], axis=1)
        hit = (iota == id0).astype(jnp.float32) + (iota == id1).astype(jnp.float32)
        cnt = jnp.concatenate([jnp.sum(hit, axis=0, keepdims=True), pad], axis=1)

        @pl.when(i == 0)
        def _():
            acc_ref[...] = jnp.zeros_like(acc_ref)

        acc_ref[0:1, :] += psum
        acc_ref[1:2, :] += cnt

        @pl.when(i == nb - 1)
        def _():
            loss_ref[0, 0] = (jnp.sum(acc_ref[0:1, :] * acc_ref[1:2, :])
                              * E / (T * T))

    return pl.pallas_call(
        body,
        grid=(nb,),
        in_specs=[
            pl.BlockSpec((_TBR, H), lambda i: (i, 0)),
            pl.BlockSpec((H, E), lambda i: (0, 0)),
        ],
        out_specs=[
            pl.BlockSpec((_TBR, 1), lambda i: (i, 0)),
            pl.BlockSpec((_TBR, 1), lambda i: (i, 0)),
            pl.BlockSpec((_TBR, 1), lambda i: (i, 0)),
            pl.BlockSpec((_TBR, 1), lambda i: (i, 0)),
            pl.BlockSpec((1, 1), lambda i: (0, 0), memory_space=pltpu.SMEM),
        ],
        out_shape=[
            jax.ShapeDtypeStruct((T, 1), jnp.int32),
            jax.ShapeDtypeStruct((T, 1), jnp.int32),
            jax.ShapeDtypeStruct((T, 1), jnp.float32),
            jax.ShapeDtypeStruct((T, 1), jnp.float32),
            jax.ShapeDtypeStruct((1, 1), jnp.float32),
        ],
        scratch_shapes=[pltpu.VMEM((8, 128), jnp.float32)],
    )(x, W_router)


def _sc_gather_rows(table, idx):
    R = idx.shape[0]
    H = table.shape[1]
    per = R // _NW
    nch = per // _CH
    mesh = plsc.VectorSubcoreMesh(core_axis_name="c", subcore_axis_name="s")

    @functools.partial(
        pl.kernel,
        out_type=jax.ShapeDtypeStruct((R, H), jnp.float32),
        mesh=mesh,
        scratch_types=[
            pltpu.VMEM((per,), jnp.int32),
            pltpu.VMEM((2, _CH, H), jnp.float32),
            pltpu.SemaphoreType.DMA,
            pltpu.SemaphoreType.DMA,
            pltpu.SemaphoreType.DMA,
            pltpu.SemaphoreType.DMA,
        ],
    )
    def k(idx_hbm, tab_hbm, out_hbm, idx_v, rows_v, gs0, gs1, ws0, ws1):
        gsems = (gs0, gs1)
        wsems = (ws0, ws1)
        wid = lax.axis_index("s") * 2 + lax.axis_index("c")
        base = wid * per
        pltpu.sync_copy(idx_hbm.at[pl.ds(base, per)], idx_v)
        gh = [None, None]
        wh = [None, None]
        gh[0] = pltpu.async_copy(tab_hbm.at[idx_v.at[pl.ds(0, _CH)]],
                                 rows_v.at[0], gs0)
        for c in range(nch):
            b = c % 2
            nb = (c + 1) % 2
            if c + 1 < nch:
                if wh[nb] is not None:
                    wh[nb].wait()
                gh[nb] = pltpu.async_copy(
                    tab_hbm.at[idx_v.at[pl.ds((c + 1) * _CH, _CH)]],
                    rows_v.at[nb], gsems[nb])
            gh[b].wait()
            wh[b] = pltpu.async_copy(
                rows_v.at[b], out_hbm.at[pl.ds(base + c * _CH, _CH)], wsems[b])
        for h in wh:
            if h is not None:
                h.wait()

    return k(idx, table)


def _grouped_ffn(x_sorted, gate_pad, block_expert, W_gate, W_up, W_down):
    R, H = x_sorted.shape
    E, _, I = W_gate.shape
    G = R // _BLK
    KC = I // _IB

    def body(ids_ref, x_ref, gate_ref, wg_ref, wu_ref, wd_ref, y_ref):
        kc = pl.program_id(1)
        x = x_ref[...]
        g = jnp.dot(x, wg_ref[0], preferred_element_type=jnp.float32)
        u = jnp.dot(x, wu_ref[0], preferred_element_type=jnp.float32)
        a = g * jax.nn.sigmoid(g) * u
        part = jnp.dot(a, wd_ref[0], preferred_element_type=jnp.float32)
        part = part * gate_ref[...]

        @pl.when(kc == 0)
        def _():
            y_ref[...] = part

        @pl.when(kc > 0)
        def _():
            y_ref[...] += part

    grid_spec = pltpu.PrefetchScalarGridSpec(
        num_scalar_prefetch=1,
        grid=(G, KC),
        in_specs=[
            pl.BlockSpec((_BLK, H), lambda g, kc, ids: (g, 0)),
            pl.BlockSpec((_BLK, 1), lambda g, kc, ids: (g, 0)),
            pl.BlockSpec((1, H, _IB), lambda g, kc, ids: (ids[g], 0, kc)),
            pl.BlockSpec((1, H, _IB), lambda g, kc, ids: (ids[g], 0, kc)),
            pl.BlockSpec((1, _IB, H), lambda g, kc, ids: (ids[g], kc, 0)),
        ],
        out_specs=pl.BlockSpec((_BLK, H), lambda g, kc, ids: (g, 0)),
    )
    return pl.pallas_call(
        body,
        grid_spec=grid_spec,
        out_shape=jax.ShapeDtypeStruct((R, H), jnp.float32),
    )(block_expert, x_sorted, gate_pad, W_gate, W_up, W_down)


def _pair_sum(combined, T):
    H = combined.shape[1]
    nb = T // _TBS

    def body(a_ref, b_ref, o_ref):
        o_ref[...] = a_ref[...] + b_ref[...]

    return pl.pallas_call(
        body,
        grid=(nb,),
        in_specs=[
            pl.BlockSpec((_TBS, H), lambda i: (i, 0)),
            pl.BlockSpec((_TBS, H), lambda i: (i + nb, 0)),
        ],
        out_specs=pl.BlockSpec((_TBS, H), lambda i: (i, 0)),
        out_shape=jax.ShapeDtypeStruct((T, H), jnp.float32),
    )(combined, combined)


def kernel(hidden_states, W_router, W_gate, W_up, W_down):
    B, S, H = hidden_states.shape
    E = W_router.shape[1]
    T = B * S
    A = _K * T
    G = A // _BLK + E
    R = G * _BLK

    x = hidden_states.reshape(T, H)
    id0, id1, w0, w1, loss = _router(x, W_router)

    e_flat = jnp.concatenate([id0[:, 0], id1[:, 0]])
    gate_flat = jnp.concatenate([w0[:, 0], w1[:, 0]])
    order = jnp.argsort(e_flat)
    e_sorted = e_flat[order]
    counts = jnp.bincount(e_flat, length=E)
    nrows_pad = ((counts + _BLK - 1) // _BLK) * _BLK
    zero = jnp.zeros((1,), counts.dtype)
    pstart = jnp.concatenate([zero, jnp.cumsum(nrows_pad)])[:E]
    start = jnp.concatenate([zero, jnp.cumsum(counts)])[:E]
    pp = (pstart[e_sorted] + jnp.arange(A) - start[e_sorted]).astype(jnp.int32)
    tok_pad = jnp.zeros((R,), jnp.int32).at[pp].set((order % T).astype(jnp.int32))
    gate_pad = jnp.zeros((R, 1), jnp.float32).at[pp, 0].set(gate_flat[order])
    src = jnp.zeros((A,), jnp.int32).at[order].set(pp)
    bstart = pstart // _BLK
    block_expert = (jnp.sum(jnp.arange(G)[:, None] >= bstart[None, :], axis=1)
                    .astype(jnp.int32) - 1)

    x_sorted = _sc_gather_rows(x, tok_pad)
    y_pad = _grouped_ffn(x_sorted, gate_pad, block_expert, W_gate, W_up, W_down)
    combined = _sc_gather_rows(y_pad, src)
    out = _pair_sum(combined, T).reshape(B, S, H)
    return out, loss[0, 0]
